# baseline (speedup 1.0000x reference)
import jax
import jax.numpy as jnp
from jax import lax
from jax.experimental import pallas as pl
from jax.experimental.pallas import tpu as pltpu
from jax.experimental.pallas import tpu_sc as plsc

NC = 2
NS = 16
NW = NC * NS
L = 16

B, S, D = 2, 8192, 4096
NWIN = 64
OUT_TOTAL = B * S * NWIN
OUT_W = OUT_TOTAL // NW
NDMA = 8
CHUNK = OUT_W // NDMA
GROUPS = CHUNK // L


def _sc_window_select(xt, w_hbm, out, w_v, idx_v, obuf, sem):
    wid = lax.axis_index("s") * NC + lax.axis_index("c")
    wbase = wid * OUT_W

    pltpu.sync_copy(w_hbm, w_v)

    wvec = [w_v[pl.ds(t * L, L)] for t in range(4)]
    xbase = [v + wbase * 64 for v in wvec]

    def fill(blk, carry):
        b0 = blk * 4096
        off = blk * 64
        for t in range(4):
            idx_v[pl.ds(off + t * L, L)] = xbase[t] + b0
        return carry

    lax.fori_loop(0, OUT_W // 64, fill, 0)

    def dma(k):
        sl = pl.ds(k * CHUNK, CHUNK)
        return pltpu.make_async_copy(xt.at[idx_v.at[sl]], obuf.at[sl], sem)

    for k in range(NDMA):
        dma(k).start()
    for k in range(NDMA):
        dma(k).wait()

    pltpu.sync_copy(obuf, out.at[wid])


@jax.jit
def kernel(x, w):
    xt = x.reshape(B * S * D)
    w32 = w.astype(jnp.int32)
    run = pl.kernel(
        _sc_window_select,
        out_type=jax.ShapeDtypeStruct((NW, OUT_W), jnp.float32),
        mesh=plsc.VectorSubcoreMesh(core_axis_name="c", subcore_axis_name="s"),
        scratch_types=[
            pltpu.VMEM((NWIN,), jnp.int32),
            pltpu.VMEM((OUT_W,), jnp.int32),
            pltpu.VMEM((OUT_W,), jnp.float32),
            pltpu.SemaphoreType.DMA,
        ],
    )
    out = run(xt, w32)
    return out.reshape(B, S, NWIN)
